# Initial kernel scaffold; baseline (speedup 1.0000x reference)
import jax
import jax.numpy as jnp
from jax.experimental import pallas as pl


def kernel(left_to_right, right_to_left, top_to_bottom, bottom_to_top, top_left_to_bottom_right, bottom_right_to_top_left, top_right_to_bottom_left, bottom_left_to_top_right):
    raise NotImplementedError("write your pallas kernel here")


# speedup vs baseline: 1.1854x; 1.1854x over previous
import functools

import numpy as np
import jax
import jax.numpy as jnp
from jax import lax
from jax.experimental import pallas as pl
from jax.experimental.pallas import tpu as pltpu
from jax.experimental.pallas import tpu_sc as plsc

_B, _HW, _C, _G = 32, 576, 768, 24
_M = _HW * _C // _G


def _diag5(H, W):
    idx = []
    for s in range(H + W - 1):
        for y in range(max(0, s - W + 1), min(H, s + 1)):
            x = s - y
            idx.append(y * W + x)
    return np.array(idx)


def _diag6(H, W):
    idx = []
    for s in range(H + W - 1):
        for x in range(min(W - 1, s), max(-1, s - H), -1):
            y = s - x
            if y < H and x < W:
                idx.append(y * W + x)
    idx.reverse()
    return np.array(idx)


def _diag78(H, W):
    idx = []
    for s in range(W + H - 1):
        for y in range(max(0, s - W + 1), min(H, s + 1)):
            x = s - y
            idx.append(y * W + (W - 1 - x))
    return np.array(idx)


_REV = np.arange(_HW - 1, -1, -1)
_PERMS = np.stack([
    np.arange(_HW),
    _REV,
    np.argsort(_diag5(_G, _G)),
    np.argsort(_diag6(_G, _G)),
    np.argsort(_diag78(_G, _G)),
    np.argsort(_diag78(_G, _G))[::-1],
    _REV,
])

_NSTREAMS = 7
_R = 16
_NCHUNK = _HW // _R

_IDX_NP = (
    np.arange(_B)[None, :, None, None] * _HW
    + _PERMS.reshape(_NSTREAMS, 1, _NCHUNK, _R)
).astype(np.int32)


def _sc_body(x1, x2, x5, x6, x7, x8, x4, idx_hbm, s_out, y4_out,
             ivec, g0, g1, g2, g3, g4, g5, g6, sem):
    w = lax.axis_index("s") * 2 + lax.axis_index("c")
    tables = (x1, x2, x5, x6, x7, x8, x4)
    bufs = (g0, g1, g2, g3, g4, g5, g6)

    def chunk(j, _):
        pltpu.sync_copy(idx_hbm.at[w, j], ivec)
        handles = []
        for i in range(_NSTREAMS):
            handles.append(
                pltpu.async_copy(tables[i].at[ivec[i]], bufs[i], sem))
        for h in handles:
            h.wait()

        def row(p, _):
            def col(k, _):
                ds = pl.ds(k * 16, 16)
                g0[p, ds] = (
                    g0[p, ds] + g1[p, ds] + g2[p, ds]
                    + g3[p, ds] + g4[p, ds] + g5[p, ds]
                )
                return _
            return lax.fori_loop(0, _C // 16, col, _)

        lax.fori_loop(0, _R, row, None)
        pltpu.sync_copy(g0, s_out.at[w, pl.ds(j * _R, _R)])
        pltpu.sync_copy(g6, y4_out.at[w, pl.ds(j * _R, _R)])
        return _

    lax.fori_loop(0, _NCHUNK, chunk, None)


def _sc_stage(x1, x2, x5, x6, x7, x8, x4):
    idx = jnp.asarray(np.ascontiguousarray(_IDX_NP.transpose(1, 2, 0, 3)))
    mesh = plsc.VectorSubcoreMesh(core_axis_name="c", subcore_axis_name="s",
                                  num_cores=2, num_subcores=16)
    f = pl.kernel(
        _sc_body,
        out_type=[
            jax.ShapeDtypeStruct((_B, _HW, _C), jnp.float32),
            jax.ShapeDtypeStruct((_B, _HW, _C), jnp.float32),
        ],
        mesh=mesh,
        scratch_types=[
            pltpu.VMEM((_NSTREAMS, _R), jnp.int32),
        ] + [pltpu.VMEM((_R, _C), jnp.float32)] * _NSTREAMS + [
            pltpu.SemaphoreType.DMA,
        ],
    )
    flat = lambda x: x.reshape(_B * _HW, _C)
    return f(flat(x1), flat(x2), flat(x5), flat(x6), flat(x7), flat(x8),
             flat(x4), idx)


def _tca_body(s_ref, out_ref):
    out_ref[0] = s_ref[0].T


def _tcb_body(prev_ref, x3_ref, y4_ref, out_ref):
    out_ref[0] = prev_ref[0] + x3_ref[0].T + y4_ref[0].T


def kernel(left_to_right, right_to_left, top_to_bottom, bottom_to_top,
           top_left_to_bottom_right, bottom_right_to_top_left,
           top_right_to_bottom_left, bottom_left_to_top_right):
    b = _B
    s, y4f = _sc_stage(
        left_to_right, right_to_left,
        top_left_to_bottom_right, bottom_right_to_top_left,
        top_right_to_bottom_left, bottom_left_to_top_right,
        bottom_to_top)

    out1 = pl.pallas_call(
        _tca_body,
        grid=(b,),
        in_specs=[pl.BlockSpec((1, _HW, _C), lambda i: (i, 0, 0))],
        out_specs=pl.BlockSpec((1, _C, _HW), lambda i: (i, 0, 0)),
        out_shape=jax.ShapeDtypeStruct((b, _C, _HW), jnp.float32),
    )(s)

    nsplit = 4
    mblk = _M // nsplit
    out2 = pl.pallas_call(
        _tcb_body,
        grid=(b, nsplit),
        in_specs=[
            pl.BlockSpec((1, mblk, _G), lambda i, j: (i, j, 0)),
            pl.BlockSpec((1, _G, mblk), lambda i, j: (i, 0, j)),
            pl.BlockSpec((1, _G, mblk), lambda i, j: (i, 0, j)),
        ],
        out_specs=pl.BlockSpec((1, mblk, _G), lambda i, j: (i, j, 0)),
        out_shape=jax.ShapeDtypeStruct((b, _M, _G), jnp.float32),
        input_output_aliases={0: 0},
    )(out1.reshape(b, _M, _G),
      top_to_bottom.reshape(b, _G, _M),
      y4f.reshape(b, _G, _M))

    return out2.reshape(b, _C, _G, _G)


# speedup vs baseline: 2.1382x; 1.8038x over previous
import functools

import numpy as np
import jax
import jax.numpy as jnp
from jax import lax
from jax.experimental import pallas as pl
from jax.experimental.pallas import tpu as pltpu
from jax.experimental.pallas import tpu_sc as plsc

_B, _HW, _C, _G = 32, 576, 768, 24
_CH = _C // _G


def _diag5(H, W):
    idx = []
    for s in range(H + W - 1):
        for y in range(max(0, s - W + 1), min(H, s + 1)):
            x = s - y
            idx.append(y * W + x)
    return np.array(idx)


def _diag6(H, W):
    idx = []
    for s in range(H + W - 1):
        for x in range(min(W - 1, s), max(-1, s - H), -1):
            y = s - x
            if y < H and x < W:
                idx.append(y * W + x)
    idx.reverse()
    return np.array(idx)


def _diag78(H, W):
    idx = []
    for s in range(W + H - 1):
        for y in range(max(0, s - W + 1), min(H, s + 1)):
            x = s - y
            idx.append(y * W + (W - 1 - x))
    return np.array(idx)


_PERMS = np.stack([
    np.arange(_HW),
    np.arange(_HW - 1, -1, -1),
    np.argsort(_diag5(_G, _G)),
    np.argsort(_diag6(_G, _G)),
    np.argsort(_diag78(_G, _G)),
    np.argsort(_diag78(_G, _G))[::-1],
])

_NS = 6
_R = 16
_NCHUNK = _HW // _R

_IDX_NP = np.ascontiguousarray(
    (np.arange(_B)[None, :, None, None] * _HW
     + _PERMS.reshape(_NS, 1, _NCHUNK, _R)).astype(np.int32)
    .transpose(1, 2, 0, 3))


def _lane_perm(flip):
    p = np.zeros((_C, _C), np.float32)
    for c in range(_CH):
        for b in range(_G):
            bb = (_G - 1 - b) if flip else b
            p[c * _G + b, bb * _CH + c] = 1.0
    return p


_P_T2B = _lane_perm(False)
_P_B2T = _lane_perm(True)


def _sc_body(x1, x2, x5, x6, x7, x8, idx_hbm, s_out,
             ivec, g0, g1, g2, g3, g4, g5, sem):
    w = lax.axis_index("s") * 2 + lax.axis_index("c")
    tables = (x1, x2, x5, x6, x7, x8)
    bufs = (g0, g1, g2, g3, g4, g5)

    def chunk(j, carry):
        pltpu.sync_copy(idx_hbm.at[w, j], ivec)
        handles = []
        for i in range(_NS):
            handles.append(
                pltpu.async_copy(tables[i].at[ivec[i]], bufs[i], sem))
        for h in handles:
            h.wait()

        def row(p, c2):
            def col(k, c3):
                ds = pl.ds(k * 16, 16)
                g0[p, ds] = (
                    g0[p, ds] + g1[p, ds] + g2[p, ds]
                    + g3[p, ds] + g4[p, ds] + g5[p, ds]
                )
                return c3
            return lax.fori_loop(0, _C // 16, col, c2)

        lax.fori_loop(0, _R, row, None)
        pltpu.sync_copy(g0, s_out.at[w, pl.ds(j * _R, _R)])
        return carry

    lax.fori_loop(0, _NCHUNK, chunk, None)


def _sc_stage(x1, x2, x5, x6, x7, x8):
    idx = jnp.asarray(_IDX_NP)
    mesh = plsc.VectorSubcoreMesh(core_axis_name="c", subcore_axis_name="s",
                                  num_cores=2, num_subcores=16)
    f = pl.kernel(
        _sc_body,
        out_type=jax.ShapeDtypeStruct((_B, _HW, _C), jnp.float32),
        mesh=mesh,
        scratch_types=[
            pltpu.VMEM((_NS, _R), jnp.int32),
        ] + [pltpu.VMEM((_R, _C), jnp.float32)] * _NS + [
            pltpu.SemaphoreType.DMA,
        ],
    )
    flat = lambda x: x.reshape(_B * _HW, _C)
    return f(flat(x1), flat(x2), flat(x5), flat(x6), flat(x7), flat(x8), idx)


def _slab_term(src, pmat, flip):
    slabs = []
    for k in range(_G):
        kk = (_G - 1 - k) if flip else k
        a = src[kk * _G:(kk + 1) * _G, :].astype(jnp.bfloat16)
        t1 = a.T
        t2 = t1.reshape(_CH, _G, _G)
        t3 = t2.transpose(0, 2, 1)
        t4 = t3.reshape(_C, _G)
        t5 = t4.T
        slabs.append(t5)
    w = jnp.stack(slabs, axis=1).reshape(_HW, _C)
    return jax.lax.dot_general(
        w, pmat, (((1,), (0,)), ((), ())),
        preferred_element_type=jnp.float32)


def _tc_body(p3_ref, p4_ref, v6_ref, x3_ref, x4_ref, out_ref):
    acc = v6_ref[0]
    acc += _slab_term(x3_ref[0], p3_ref[...], False)
    acc += _slab_term(x4_ref[0], p4_ref[...], True)
    out_ref[0] = acc


def kernel(left_to_right, right_to_left, top_to_bottom, bottom_to_top,
           top_left_to_bottom_right, bottom_right_to_top_left,
           top_right_to_bottom_left, bottom_left_to_top_right):
    b = _B
    v6 = _sc_stage(
        left_to_right, right_to_left,
        top_left_to_bottom_right, bottom_right_to_top_left,
        top_right_to_bottom_left, bottom_left_to_top_right)

    p3 = jnp.asarray(_P_T2B, dtype=jnp.bfloat16)
    p4 = jnp.asarray(_P_B2T, dtype=jnp.bfloat16)

    big = pl.BlockSpec((1, _HW, _C), lambda i: (i, 0, 0))
    out = pl.pallas_call(
        _tc_body,
        grid=(b,),
        in_specs=[
            pl.BlockSpec((_C, _C), lambda i: (0, 0)),
            pl.BlockSpec((_C, _C), lambda i: (0, 0)),
            big, big, big,
        ],
        out_specs=big,
        out_shape=jax.ShapeDtypeStruct((b, _HW, _C), jnp.float32),
    )(p3, p4, v6, top_to_bottom, bottom_to_top)

    return out.reshape(b, _G, _G, _C).transpose(0, 3, 1, 2)


# speedup vs baseline: 2.2120x; 1.0345x over previous
import functools

import numpy as np
import jax
import jax.numpy as jnp
from jax import lax
from jax.experimental import pallas as pl
from jax.experimental.pallas import tpu as pltpu
from jax.experimental.pallas import tpu_sc as plsc

_B, _HW, _C, _G = 32, 576, 768, 24
_CH = _C // _G


def _diag5(H, W):
    idx = []
    for s in range(H + W - 1):
        for y in range(max(0, s - W + 1), min(H, s + 1)):
            x = s - y
            idx.append(y * W + x)
    return np.array(idx)


def _diag6(H, W):
    idx = []
    for s in range(H + W - 1):
        for x in range(min(W - 1, s), max(-1, s - H), -1):
            y = s - x
            if y < H and x < W:
                idx.append(y * W + x)
    idx.reverse()
    return np.array(idx)


def _diag78(H, W):
    idx = []
    for s in range(W + H - 1):
        for y in range(max(0, s - W + 1), min(H, s + 1)):
            x = s - y
            idx.append(y * W + (W - 1 - x))
    return np.array(idx)


_PERMS = np.stack([
    np.arange(_HW),
    np.arange(_HW - 1, -1, -1),
    np.argsort(_diag5(_G, _G)),
    np.argsort(_diag6(_G, _G)),
    np.argsort(_diag78(_G, _G)),
    np.argsort(_diag78(_G, _G))[::-1],
])

_NS = 6
_R = 16
_NCHUNK = _HW // _R

_IDX_NP = np.ascontiguousarray(
    (np.arange(_B)[None, :, None, None] * _HW
     + _PERMS.reshape(_NS, 1, _NCHUNK, _R)).astype(np.int32)
    .transpose(1, 2, 0, 3))


def _lane_perm():
    p = np.zeros((_C, _C), np.float32)
    for c in range(_CH):
        for b in range(_G):
            p[c * _G + b, b * _CH + c] = 1.0
    return p


_P_T2B = _lane_perm()
_REV_NP = np.eye(_HW, dtype=np.float32)[::-1].copy()


def _sc_body(x1, x2, x5, x6, x7, x8, idx_hbm, s_out,
             ivec, g0, g1, g2, g3, g4, g5, sem):
    w = lax.axis_index("s") * 2 + lax.axis_index("c")
    tables = (x1, x2, x5, x6, x7, x8)
    bufs = (g0, g1, g2, g3, g4, g5)
    pltpu.sync_copy(idx_hbm.at[w], ivec)

    def chunk(j, carry):
        handles = []
        for i in range(_NS):
            handles.append(
                pltpu.async_copy(tables[i].at[ivec[j, i]], bufs[i], sem))
        for h in handles:
            h.wait()

        def row(p, c2):
            def col(k, c3):
                for u in range(4):
                    ds = pl.ds((k * 4 + u) * 16, 16)
                    g0[p, ds] = (
                        g0[p, ds] + g1[p, ds] + g2[p, ds]
                        + g3[p, ds] + g4[p, ds] + g5[p, ds]
                    )
                return c3
            return lax.fori_loop(0, _C // 64, col, c2)

        lax.fori_loop(0, _R, row, None)
        pltpu.sync_copy(g0, s_out.at[w, pl.ds(j * _R, _R)])
        return carry

    lax.fori_loop(0, _NCHUNK, chunk, None)


def _sc_stage(x1, x2, x5, x6, x7, x8):
    idx = jnp.asarray(_IDX_NP)
    mesh = plsc.VectorSubcoreMesh(core_axis_name="c", subcore_axis_name="s",
                                  num_cores=2, num_subcores=16)
    f = pl.kernel(
        _sc_body,
        out_type=jax.ShapeDtypeStruct((_B, _HW, _C), jnp.float32),
        mesh=mesh,
        scratch_types=[
            pltpu.VMEM((_NCHUNK, _NS, _R), jnp.int32),
        ] + [pltpu.VMEM((_R, _C), jnp.float32)] * _NS + [
            pltpu.SemaphoreType.DMA,
        ],
    )
    flat = lambda x: x.reshape(_B * _HW, _C)
    return f(flat(x1), flat(x2), flat(x5), flat(x6), flat(x7), flat(x8), idx)


def _t2b_w(ybf):
    x3d = ybf.reshape(_G, _G, _C)
    w1 = x3d.transpose(0, 2, 1)
    w2 = w1.reshape(_G, _CH, _G, _G)
    w3 = w2.transpose(0, 1, 3, 2)
    w4 = w3.reshape(_G, _C, _G)
    w5 = w4.transpose(0, 2, 1)
    w6 = w5.transpose(1, 0, 2)
    return w6.reshape(_HW, _C)


def _tc_body(p3_ref, rev_ref, v6_ref, x3_ref, x4_ref, out_ref):
    x3b = x3_ref[0].astype(jnp.bfloat16)
    x4b = x4_ref[0].astype(jnp.bfloat16)
    x4f = jax.lax.dot_general(
        rev_ref[...], x4b, (((1,), (0,)), ((), ())),
        preferred_element_type=jnp.float32).astype(jnp.bfloat16)
    w = jnp.concatenate([_t2b_w(x3b), _t2b_w(x4f)], axis=1)
    p2 = jnp.concatenate([p3_ref[...], p3_ref[...]], axis=0)
    term = jax.lax.dot_general(
        w, p2, (((1,), (0,)), ((), ())),
        preferred_element_type=jnp.float32)
    out_ref[0] = v6_ref[0] + term


def kernel(left_to_right, right_to_left, top_to_bottom, bottom_to_top,
           top_left_to_bottom_right, bottom_right_to_top_left,
           top_right_to_bottom_left, bottom_left_to_top_right):
    b = _B
    v6 = _sc_stage(
        left_to_right, right_to_left,
        top_left_to_bottom_right, bottom_right_to_top_left,
        top_right_to_bottom_left, bottom_left_to_top_right)

    p3 = jnp.asarray(_P_T2B, dtype=jnp.bfloat16)
    rev = jnp.asarray(_REV_NP, dtype=jnp.bfloat16)

    big = pl.BlockSpec((1, _HW, _C), lambda i: (i, 0, 0))
    out = pl.pallas_call(
        _tc_body,
        grid=(b,),
        in_specs=[
            pl.BlockSpec((_C, _C), lambda i: (0, 0)),
            pl.BlockSpec((_HW, _HW), lambda i: (0, 0)),
            big, big, big,
        ],
        out_specs=big,
        out_shape=jax.ShapeDtypeStruct((b, _HW, _C), jnp.float32),
    )(p3, rev, v6, top_to_bottom, bottom_to_top)

    return out.reshape(b, _G, _G, _C).transpose(0, 3, 1, 2)


# speedup vs baseline: 2.9230x; 1.3214x over previous
import functools

import numpy as np
import jax
import jax.numpy as jnp
from jax import lax
from jax.experimental import pallas as pl
from jax.experimental.pallas import tpu as pltpu
from jax.experimental.pallas import tpu_sc as plsc

_B, _HW, _C, _G = 32, 576, 768, 24
_CH = _C // _G


def _diag5(H, W):
    idx = []
    for s in range(H + W - 1):
        for y in range(max(0, s - W + 1), min(H, s + 1)):
            x = s - y
            idx.append(y * W + x)
    return np.array(idx)


def _diag6(H, W):
    idx = []
    for s in range(H + W - 1):
        for x in range(min(W - 1, s), max(-1, s - H), -1):
            y = s - x
            if y < H and x < W:
                idx.append(y * W + x)
    idx.reverse()
    return np.array(idx)


def _diag78(H, W):
    idx = []
    for s in range(W + H - 1):
        for y in range(max(0, s - W + 1), min(H, s + 1)):
            x = s - y
            idx.append(y * W + (W - 1 - x))
    return np.array(idx)


_PERMS = np.stack([
    np.argsort(_diag5(_G, _G)),
    np.argsort(_diag6(_G, _G)),
    np.argsort(_diag78(_G, _G)),
    np.argsort(_diag78(_G, _G))[::-1],
])

_NS = 4
_R = 16
_NCHUNK = _HW // _R

_IDX_NP = np.ascontiguousarray(
    (np.arange(_B)[None, :, None, None] * _HW
     + _PERMS.reshape(_NS, 1, _NCHUNK, _R)).astype(np.int32)
    .transpose(1, 2, 0, 3))


def _lane_perm():
    p = np.zeros((_C, _C), np.float32)
    for c in range(_CH):
        for b in range(_G):
            p[c * _G + b, b * _CH + c] = 1.0
    return p


_P_T2B = _lane_perm()
_REV_NP = np.eye(_HW, dtype=np.float32)[::-1].copy()


def _sc_body(x5, x6, x7, x8, idx_hbm, s_out,
             ivec, a0, a1, a2, a3, b0, b1, b2, b3, sema, semb):
    w = lax.axis_index("s") * 2 + lax.axis_index("c")
    tables = (x5, x6, x7, x8)
    seta = (a0, a1, a2, a3)
    setb = (b0, b1, b2, b3)
    pltpu.sync_copy(idx_hbm.at[w], ivec)

    def fire(bufs, sem, j):
        for i in range(_NS):
            pltpu.async_copy(tables[i].at[ivec[j, i]], bufs[i], sem)

    def drain(bufs, sem):
        for i in range(_NS):
            pltpu.make_async_copy(tables[i].at[ivec[0, i]], bufs[i],
                                  sem).wait()

    def consume(bufs, j):
        g0, g1, g2, g3 = bufs

        def row(p, c2):
            def col(k, c3):
                for u in range(4):
                    ds = pl.ds((k * 4 + u) * 16, 16)
                    g0[p, ds] = g0[p, ds] + g1[p, ds] + g2[p, ds] + g3[p, ds]
                return c3
            return lax.fori_loop(0, _C // 64, col, c2)

        lax.fori_loop(0, _R, row, None)
        pltpu.sync_copy(g0, s_out.at[w, pl.ds(j * _R, _R)])

    fire(seta, sema, 0)

    def pair(jj, carry):
        j0 = jj * 2
        fire(setb, semb, j0 + 1)
        drain(seta, sema)
        consume(seta, j0)

        @pl.when(jj + 1 < _NCHUNK // 2)
        def _():
            fire(seta, sema, j0 + 2)
        drain(setb, semb)
        consume(setb, j0 + 1)
        return carry

    lax.fori_loop(0, _NCHUNK // 2, pair, None)


def _sc_stage(x5, x6, x7, x8):
    idx = jnp.asarray(_IDX_NP)
    mesh = plsc.VectorSubcoreMesh(core_axis_name="c", subcore_axis_name="s",
                                  num_cores=2, num_subcores=16)
    f = pl.kernel(
        _sc_body,
        out_type=jax.ShapeDtypeStruct((_B, _HW, _C), jnp.float32),
        mesh=mesh,
        scratch_types=[
            pltpu.VMEM((_NCHUNK, _NS, _R), jnp.int32),
        ] + [pltpu.VMEM((_R, _C), jnp.float32)] * (2 * _NS) + [
            pltpu.SemaphoreType.DMA,
            pltpu.SemaphoreType.DMA,
        ],
    )
    flat = lambda x: x.reshape(_B * _HW, _C)
    return f(flat(x5), flat(x6), flat(x7), flat(x8), idx)


def _t2b_w(ybf):
    x3d = ybf.reshape(_G, _G, _C)
    w1 = x3d.transpose(0, 2, 1)
    w2 = w1.reshape(_G, _CH, _G, _G)
    w3 = w2.transpose(0, 1, 3, 2)
    w4 = w3.reshape(_G, _C, _G)
    w5 = w4.transpose(0, 2, 1)
    w6 = w5.transpose(1, 0, 2)
    return w6.reshape(_HW, _C)


def _tc_body(p3_ref, rev_ref, v4_ref, x1_ref, x2_ref, x3_ref, x4_ref,
             out_ref):
    x3b = x3_ref[0].astype(jnp.bfloat16)
    x4b = x4_ref[0].astype(jnp.bfloat16)
    x4f = jax.lax.dot_general(
        rev_ref[...], x4b, (((1,), (0,)), ((), ())),
        preferred_element_type=jnp.float32).astype(jnp.bfloat16)
    x2f = jax.lax.dot_general(
        rev_ref[...], x2_ref[0].astype(jnp.bfloat16), (((1,), (0,)), ((), ())),
        preferred_element_type=jnp.float32)
    w = jnp.concatenate([_t2b_w(x3b), _t2b_w(x4f)], axis=1)
    p2 = jnp.concatenate([p3_ref[...], p3_ref[...]], axis=0)
    term = jax.lax.dot_general(
        w, p2, (((1,), (0,)), ((), ())),
        preferred_element_type=jnp.float32)
    out_ref[0] = v4_ref[0] + x1_ref[0] + x2f + term


def kernel(left_to_right, right_to_left, top_to_bottom, bottom_to_top,
           top_left_to_bottom_right, bottom_right_to_top_left,
           top_right_to_bottom_left, bottom_left_to_top_right):
    b = _B
    v4 = _sc_stage(
        top_left_to_bottom_right, bottom_right_to_top_left,
        top_right_to_bottom_left, bottom_left_to_top_right)

    p3 = jnp.asarray(_P_T2B, dtype=jnp.bfloat16)
    rev = jnp.asarray(_REV_NP, dtype=jnp.bfloat16)

    big = pl.BlockSpec((1, _HW, _C), lambda i: (i, 0, 0))
    out = pl.pallas_call(
        _tc_body,
        grid=(b,),
        in_specs=[
            pl.BlockSpec((_C, _C), lambda i: (0, 0)),
            pl.BlockSpec((_HW, _HW), lambda i: (0, 0)),
            big, big, big, big, big,
        ],
        out_specs=big,
        out_shape=jax.ShapeDtypeStruct((b, _HW, _C), jnp.float32),
    )(p3, rev, v4, left_to_right, right_to_left, top_to_bottom, bottom_to_top)

    return out.reshape(b, _G, _G, _C).transpose(0, 3, 1, 2)


# speedup vs baseline: 3.6114x; 1.2355x over previous
import functools

import numpy as np
import jax
import jax.numpy as jnp
from jax import lax
from jax.experimental import pallas as pl
from jax.experimental.pallas import tpu as pltpu
from jax.experimental.pallas import tpu_sc as plsc

_B, _HW, _C, _G = 32, 576, 768, 24
_CH = _C // _G


def _diag5(H, W):
    idx = []
    for s in range(H + W - 1):
        for y in range(max(0, s - W + 1), min(H, s + 1)):
            x = s - y
            idx.append(y * W + x)
    return np.array(idx)


def _diag6(H, W):
    idx = []
    for s in range(H + W - 1):
        for x in range(min(W - 1, s), max(-1, s - H), -1):
            y = s - x
            if y < H and x < W:
                idx.append(y * W + x)
    idx.reverse()
    return np.array(idx)


def _diag78(H, W):
    idx = []
    for s in range(W + H - 1):
        for y in range(max(0, s - W + 1), min(H, s + 1)):
            x = s - y
            idx.append(y * W + (W - 1 - x))
    return np.array(idx)


_PERMS = np.stack([
    np.argsort(_diag5(_G, _G)),
    np.argsort(_diag6(_G, _G)),
    np.argsort(_diag78(_G, _G)),
    np.argsort(_diag78(_G, _G))[::-1],
])

_NS = 4
_R = 16
_NCHUNK = _HW // _R

_NCHL = _NCHUNK // 2

_IDX_NP = np.empty((2, 32, _NCHL, _NS, _R), np.int32)
for _h in range(2):
    for _w in range(32):
        _bg = _h * 16 + _w // 2
        for _j in range(_NCHL):
            _jg = (_w % 2) * _NCHL + _j
            for _i in range(_NS):
                _IDX_NP[_h, _w, _j, _i] = (
                    _bg * _HW + _PERMS[_i, _jg * _R:(_jg + 1) * _R])


def _lane_perm():
    p = np.zeros((_C, _C), np.float32)
    for c in range(_CH):
        for b in range(_G):
            p[c * _G + b, b * _CH + c] = 1.0
    return p


_P_T2B = _lane_perm()
_REV_NP = np.eye(_HW, dtype=np.float32)[::-1].copy()


def _sc_body(x5, x6, x7, x8, idx_hbm, s_out,
             ivec, a0, a1, a2, a3, b0, b1, b2, b3, sema, semb):
    w = lax.axis_index("s") * 2 + lax.axis_index("c")
    bloc = w // 2
    roff = (w % 2) * (_NCHL * _R)
    tables = (x5, x6, x7, x8)
    seta = (a0, a1, a2, a3)
    setb = (b0, b1, b2, b3)
    pltpu.sync_copy(idx_hbm.at[w], ivec)

    def fire(bufs, sem, j):
        for i in range(_NS):
            pltpu.async_copy(tables[i].at[ivec[j, i]], bufs[i], sem)

    def drain(bufs, sem):
        for i in range(_NS):
            pltpu.make_async_copy(tables[i].at[ivec[0, i]], bufs[i],
                                  sem).wait()

    def consume(bufs, j):
        g0, g1, g2, g3 = bufs

        def row(p, c2):
            def col(k, c3):
                for u in range(4):
                    ds = pl.ds((k * 4 + u) * 16, 16)
                    g0[p, ds] = g0[p, ds] + g1[p, ds] + g2[p, ds] + g3[p, ds]
                return c3
            return lax.fori_loop(0, _C // 64, col, c2)

        lax.fori_loop(0, _R, row, None)
        pltpu.sync_copy(g0, s_out.at[bloc, pl.ds(roff + j * _R, _R)])

    fire(seta, sema, 0)

    def pair(jj, carry):
        j0 = jj * 2
        fire(setb, semb, j0 + 1)
        drain(seta, sema)
        consume(seta, j0)

        @pl.when(jj + 1 < _NCHL // 2)
        def _():
            fire(seta, sema, j0 + 2)
        drain(setb, semb)
        consume(setb, j0 + 1)
        return carry

    lax.fori_loop(0, _NCHL // 2, pair, None)


def _sc_stage(x5, x6, x7, x8, idx_half):
    mesh = plsc.VectorSubcoreMesh(core_axis_name="c", subcore_axis_name="s",
                                  num_cores=2, num_subcores=16)
    f = pl.kernel(
        _sc_body,
        out_type=jax.ShapeDtypeStruct((_B // 2, _HW, _C), jnp.float32),
        mesh=mesh,
        scratch_types=[
            pltpu.VMEM((_NCHL, _NS, _R), jnp.int32),
        ] + [pltpu.VMEM((_R, _C), jnp.float32)] * (2 * _NS) + [
            pltpu.SemaphoreType.DMA,
            pltpu.SemaphoreType.DMA,
        ],
    )
    flat = lambda x: x.reshape(_B * _HW, _C)
    return f(flat(x5), flat(x6), flat(x7), flat(x8), idx_half)


def _t2b_w(ybf):
    x3d = ybf.reshape(_G, _G, _C)
    w1 = x3d.transpose(0, 2, 1)
    w2 = w1.reshape(_G, _CH, _G, _G)
    w3 = w2.transpose(0, 1, 3, 2)
    w4 = w3.reshape(_G, _C, _G)
    w5 = w4.transpose(0, 2, 1)
    w6 = w5.transpose(1, 0, 2)
    return w6.reshape(_HW, _C)


def _tc_body_alias(p3_ref, rev_ref, v4_ref, x1_ref, x2_ref, x3_ref, x4_ref,
                   prev_ref, out_ref):
    _tc_body(p3_ref, rev_ref, v4_ref, x1_ref, x2_ref, x3_ref, x4_ref,
             out_ref)


def _tc_body(p3_ref, rev_ref, v4_ref, x1_ref, x2_ref, x3_ref, x4_ref,
             out_ref):
    x3b = x3_ref[0].astype(jnp.bfloat16)
    x4b = x4_ref[0].astype(jnp.bfloat16)
    x4f = jax.lax.dot_general(
        rev_ref[...], x4b, (((1,), (0,)), ((), ())),
        preferred_element_type=jnp.float32).astype(jnp.bfloat16)
    x2f = jax.lax.dot_general(
        rev_ref[...], x2_ref[0].astype(jnp.bfloat16), (((1,), (0,)), ((), ())),
        preferred_element_type=jnp.float32)
    w = jnp.concatenate([_t2b_w(x3b), _t2b_w(x4f)], axis=1)
    p2 = jnp.concatenate([p3_ref[...], p3_ref[...]], axis=0)
    term = jax.lax.dot_general(
        w, p2, (((1,), (0,)), ((), ())),
        preferred_element_type=jnp.float32)
    out_ref[0] = v4_ref[0] + x1_ref[0] + x2f + term


def kernel(left_to_right, right_to_left, top_to_bottom, bottom_to_top,
           top_left_to_bottom_right, bottom_right_to_top_left,
           top_right_to_bottom_left, bottom_left_to_top_right):
    b = _B
    diag = (top_left_to_bottom_right, bottom_right_to_top_left,
            top_right_to_bottom_left, bottom_left_to_top_right)
    idx = jnp.asarray(_IDX_NP)
    v4a = _sc_stage(*diag, idx[0])
    v4b = _sc_stage(*diag, idx[1])

    p3 = jnp.asarray(_P_T2B, dtype=jnp.bfloat16)
    rev = jnp.asarray(_REV_NP, dtype=jnp.bfloat16)

    out_shape = jax.ShapeDtypeStruct((b, _HW, _C), jnp.float32)
    out = None
    for half, v4 in ((0, v4a), (1, v4b)):
        off = half * (b // 2)
        big = pl.BlockSpec((1, _HW, _C), lambda i, o=off: (i + o, 0, 0))
        loc = pl.BlockSpec((1, _HW, _C), lambda i: (i, 0, 0))
        specs = [
            pl.BlockSpec((_C, _C), lambda i: (0, 0)),
            pl.BlockSpec((_HW, _HW), lambda i: (0, 0)),
            loc, big, big, big, big,
        ]
        args = [p3, rev, v4, left_to_right, right_to_left,
                top_to_bottom, bottom_to_top]
        body = _tc_body
        kwargs = {}
        if half == 1:
            specs.append(pl.BlockSpec(memory_space=pltpu.MemorySpace.HBM))
            args.append(out)
            body = _tc_body_alias
            kwargs["input_output_aliases"] = {7: 0}
        out = pl.pallas_call(
            body,
            grid=(b // 2,),
            in_specs=specs,
            out_specs=big,
            out_shape=out_shape,
            **kwargs,
        )(*args)

    return out.reshape(b, _G, _G, _C).transpose(0, 3, 1, 2)
